# baseline (speedup 1.0000x reference)
import functools

import jax
import jax.numpy as jnp
from jax import lax
from jax.experimental import pallas as pl
from jax.experimental.pallas import tpu as pltpu
from jax.experimental.pallas import tpu_sc as plsc

_B, _D = 16384, 64
_NC, _NS, _L = 2, 16, 16
_NW = _NC * _NS
_RPW = _B // _NW
_CH = 64
_NCH = _RPW // _CH

_mesh = plsc.VectorSubcoreMesh(core_axis_name="c", subcore_axis_name="s")


@functools.partial(
    pl.kernel,
    out_type=jax.ShapeDtypeStruct((_B,), jnp.float32),
    mesh=_mesh,
    compiler_params=pltpu.CompilerParams(needs_layout_passes=False),
    scratch_types=[
        pltpu.VMEM((2, _CH, _D), jnp.float32),
        pltpu.VMEM((2, _CH, _D), jnp.float32),
        pltpu.VMEM((2, _CH, _D), jnp.float32),
        pltpu.VMEM((2, _CH, _D), jnp.float32),
        pltpu.VMEM((_RPW,), jnp.float32),
        pltpu.VMEM((_L * _L,), jnp.float32),
        pltpu.SemaphoreType.DMA,
        pltpu.SemaphoreType.DMA,
    ],
)
def _sc_kernel(gu_h, gi_h, gut_h, git_h, out_h,
               agu, agi, agut, agit, outv, t_t, sem0, sem1):
    wid = lax.axis_index("s") * _NC + lax.axis_index("c")
    base = wid * _RPW
    sems = (sem0, sem1)
    iota = lax.iota(jnp.int32, _L)

    def start(ci, slot):
        handles = []
        for h, v in ((gu_h, agu), (gi_h, agi), (gut_h, agut), (git_h, agit)):
            handles.append(
                pltpu.async_copy(h.at[pl.ds(base + ci * _CH, _CH)],
                                 v.at[slot], sems[slot]))
        return handles

    pending = start(0, 0)

    for ci in range(_NCH):
        slot = ci % 2
        for hnd in pending:
            hnd.wait()
        if ci + 1 < _NCH:
            pending = start(ci + 1, (ci + 1) % 2)
        a = agu.at[slot]
        b = agi.at[slot]
        c = agut.at[slot]
        d = agit.at[slot]

        def group(g, _):
            row0 = g * _L
            for r in range(_L):
                row = row0 + r
                q = None
                for j in range(_D // _L):
                    sl = pl.ds(j * _L, _L)
                    t = a[row, sl] * b[row, sl] + c[row, sl] * d[row, sl]
                    q = t if q is None else q + t
                t_t[pl.ds(r * _L, _L)] = q
            acc = plsc.load_gather(t_t, [iota * _L])
            for cc in range(1, _L):
                acc = acc + plsc.load_gather(t_t, [iota * _L + cc])
            outv[pl.ds(ci * _CH + row0, _L)] = acc
            return 0

        lax.fori_loop(0, _CH // _L, group, 0)

    pltpu.sync_copy(outv, out_h.at[pl.ds(base, _RPW)])


def kernel(gu, gi, gut, git):
    return _sc_kernel(gu, gi, gut, git)
